# Initial kernel scaffold; baseline (speedup 1.0000x reference)
import jax
import jax.numpy as jnp
from jax.experimental import pallas as pl


def kernel(input, encoder_weight, glove_weight):
    raise NotImplementedError("write your pallas kernel here")


# speedup vs baseline: 5.3175x; 5.3175x over previous
import functools

import jax
import jax.numpy as jnp
from jax import lax
from jax.experimental import pallas as pl
from jax.experimental.pallas import tpu as pltpu
from jax.experimental.pallas import tpu_sc as plsc

NTOKEN = 100000
D = 64
B = 16384
L = 50
N = B * L
NC = 2
NS = 16
NW = NC * NS
CHUNK = 128
PER_W = N // NW
NSTEPS = PER_W // CHUNK
LANES = 16

_mesh = plsc.VectorSubcoreMesh(core_axis_name="c", subcore_axis_name="s")


@functools.partial(
    pl.kernel,
    out_type=(
        jax.ShapeDtypeStruct((N, D), jnp.float32),
        jax.ShapeDtypeStruct((N, D), jnp.float32),
        jax.ShapeDtypeStruct((NW, LANES), jnp.float32),
    ),
    mesh=_mesh,
    compiler_params=pltpu.CompilerParams(use_tc_tiling_on_sc=False),
    scratch_types=[
        pltpu.VMEM((NSTEPS, CHUNK), jnp.int32),
        pltpu.VMEM((CHUNK, D), jnp.float32),
        pltpu.VMEM((CHUNK, D), jnp.float32),
        pltpu.VMEM((LANES,), jnp.float32),
        pltpu.SemaphoreType.DMA,
    ],
)
def _sc_gather(idx_hbm, enc_hbm, glv_hbm, out_e, out_g, out_p,
               idx_v, e_v, g_v, acc_v, sem):
    wid = lax.axis_index("s") * NC + lax.axis_index("c")
    row0 = wid * PER_W
    pltpu.sync_copy(idx_hbm.at[pl.ds(wid * NSTEPS, NSTEPS)], idx_v)

    def chunk_body(k, accs):
        pltpu.async_copy(enc_hbm.at[idx_v.at[k]], e_v, sem).wait()
        pltpu.async_copy(glv_hbm.at[idx_v.at[k]], g_v, sem).wait()

        def row_body(i, accs):
            a0, a1, a2, a3 = accs
            d0 = e_v[i, pl.ds(0, LANES)] - g_v[i, pl.ds(0, LANES)]
            d1 = e_v[i, pl.ds(16, LANES)] - g_v[i, pl.ds(16, LANES)]
            d2 = e_v[i, pl.ds(32, LANES)] - g_v[i, pl.ds(32, LANES)]
            d3 = e_v[i, pl.ds(48, LANES)] - g_v[i, pl.ds(48, LANES)]
            return (a0 + d0 * d0, a1 + d1 * d1, a2 + d2 * d2, a3 + d3 * d3)

        accs = lax.fori_loop(0, CHUNK, row_body, accs)
        dst = pl.ds(row0 + k * CHUNK, CHUNK)
        pltpu.sync_copy(e_v, out_e.at[dst])
        pltpu.sync_copy(g_v, out_g.at[dst])
        return accs

    zero = jnp.zeros((LANES,), jnp.float32)
    a0, a1, a2, a3 = lax.fori_loop(0, NSTEPS, chunk_body, (zero, zero, zero, zero))
    acc_v[...] = (a0 + a1) + (a2 + a3)
    pltpu.sync_copy(acc_v, out_p.at[wid])


def _tc_sum_body(p_ref, o_ref):
    o_ref[0, 0] = jnp.sum(p_ref[...]) * jnp.float32(1.0 / (N * D))


_tc_sum = pl.pallas_call(
    _tc_sum_body,
    out_shape=jax.ShapeDtypeStruct((1, 1), jnp.float32),
    out_specs=pl.BlockSpec(memory_space=pltpu.SMEM),
)


def kernel(input, encoder_weight, glove_weight):
    idx = input.reshape(N // CHUNK, CHUNK).astype(jnp.int32)
    emb, emb_glove, parts = _sc_gather(idx, encoder_weight, glove_weight)
    glove_loss = _tc_sum(parts)[0, 0]
    return (emb.reshape(B, L, D), emb_glove.reshape(B, L, D), glove_loss)


# speedup vs baseline: 6.8271x; 1.2839x over previous
import functools

import jax
import jax.numpy as jnp
from jax import lax
from jax.experimental import pallas as pl
from jax.experimental.pallas import tpu as pltpu
from jax.experimental.pallas import tpu_sc as plsc

NTOKEN = 100000
D = 64
B = 16384
L = 50
N = B * L
NC = 2
NS = 16
NW = NC * NS
CHUNK = 128
PER_W = N // NW
NSTEPS = PER_W // CHUNK
GROUPS = NSTEPS // 4
LANES = 16

_mesh = plsc.VectorSubcoreMesh(core_axis_name="c", subcore_axis_name="s")


@functools.partial(
    pl.kernel,
    out_type=(
        jax.ShapeDtypeStruct((N, D), jnp.float32),
        jax.ShapeDtypeStruct((N, D), jnp.float32),
        jax.ShapeDtypeStruct((NW, LANES), jnp.float32),
    ),
    mesh=_mesh,
    compiler_params=pltpu.CompilerParams(use_tc_tiling_on_sc=False),
    scratch_types=[
        pltpu.VMEM((NSTEPS, CHUNK), jnp.int32),
        [pltpu.VMEM((CHUNK, D), jnp.float32)] * 4,
        [pltpu.VMEM((CHUNK, D), jnp.float32)] * 4,
        pltpu.VMEM((LANES,), jnp.float32),
        [pltpu.SemaphoreType.DMA] * 4,
        [pltpu.SemaphoreType.DMA] * 4,
    ],
)
def _sc_gather(idx_hbm, enc_hbm, glv_hbm, out_e, out_g, out_p,
               idx_v, e_slots, g_slots, acc_v, gsems, wsems):
    wid = lax.axis_index("s") * NC + lax.axis_index("c")
    row0 = wid * PER_W
    pltpu.sync_copy(idx_hbm.at[pl.ds(wid * NSTEPS, NSTEPS)], idx_v)

    def fire_gather(k, b):
        pltpu.async_copy(enc_hbm.at[idx_v.at[k]], e_slots[b], gsems[b])
        pltpu.async_copy(glv_hbm.at[idx_v.at[k]], g_slots[b], gsems[b])

    def wait_gather(b):
        pltpu.make_async_copy(enc_hbm.at[idx_v.at[0]], e_slots[b], gsems[b]).wait()
        pltpu.make_async_copy(glv_hbm.at[idx_v.at[0]], g_slots[b], gsems[b]).wait()

    def fire_write(k, b):
        dst = pl.ds(row0 + k * CHUNK, CHUNK)
        pltpu.async_copy(e_slots[b], out_e.at[dst], wsems[b])
        pltpu.async_copy(g_slots[b], out_g.at[dst], wsems[b])

    def wait_write(b):
        pltpu.make_async_copy(e_slots[b], out_e.at[pl.ds(0, CHUNK)], wsems[b]).wait()
        pltpu.make_async_copy(g_slots[b], out_g.at[pl.ds(0, CHUNK)], wsems[b]).wait()

    def compute(b, accs):
        e_v, g_v = e_slots[b], g_slots[b]

        def row_body(i, accs):
            a0, a1, a2, a3 = accs
            d0 = e_v[i, pl.ds(0, LANES)] - g_v[i, pl.ds(0, LANES)]
            d1 = e_v[i, pl.ds(16, LANES)] - g_v[i, pl.ds(16, LANES)]
            d2 = e_v[i, pl.ds(32, LANES)] - g_v[i, pl.ds(32, LANES)]
            d3 = e_v[i, pl.ds(48, LANES)] - g_v[i, pl.ds(48, LANES)]
            return (a0 + d0 * d0, a1 + d1 * d1, a2 + d2 * d2, a3 + d3 * d3)

        return lax.fori_loop(0, CHUNK, row_body, accs)

    def service(k, b, accs, fire_next=True, drain_prev=True):
        wait_gather(b)
        accs = compute(b, accs)
        fire_write(k, b)
        if fire_next:
            b2 = (b + 2) % 4
            if drain_prev:
                wait_write(b2)
            fire_gather(k + 2, b2)
        return accs

    zero = jnp.zeros((LANES,), jnp.float32)
    accs = (zero, zero, zero, zero)

    fire_gather(0, 0)
    fire_gather(1, 1)
    accs = service(0, 0, accs, drain_prev=False)
    accs = service(1, 1, accs, drain_prev=False)
    accs = service(2, 2, accs)
    accs = service(3, 3, accs)

    def group_body(g, accs):
        k = 4 * g
        accs = service(k, 0, accs)
        accs = service(k + 1, 1, accs)
        accs = service(k + 2, 2, accs)
        accs = service(k + 3, 3, accs)
        return accs

    accs = lax.fori_loop(1, GROUPS - 1, group_body, accs)

    k = NSTEPS - 4
    accs = service(k, 0, accs)
    accs = service(k + 1, 1, accs)
    accs = service(k + 2, 2, accs, fire_next=False)
    accs = service(k + 3, 3, accs, fire_next=False)
    for b in range(4):
        wait_write(b)

    a0, a1, a2, a3 = accs
    acc_v[...] = (a0 + a1) + (a2 + a3)
    pltpu.sync_copy(acc_v, out_p.at[wid])


def _tc_sum_body(p_ref, o_ref):
    o_ref[0, 0] = jnp.sum(p_ref[...]) * jnp.float32(1.0 / (N * D))


_tc_sum = pl.pallas_call(
    _tc_sum_body,
    out_shape=jax.ShapeDtypeStruct((1, 1), jnp.float32),
    out_specs=pl.BlockSpec(memory_space=pltpu.SMEM),
)


def kernel(input, encoder_weight, glove_weight):
    idx = input.reshape(N // CHUNK, CHUNK).astype(jnp.int32)
    emb, emb_glove, parts = _sc_gather(idx, encoder_weight, glove_weight)
    glove_loss = _tc_sum(parts)[0, 0]
    return (emb.reshape(B, L, D), emb_glove.reshape(B, L, D), glove_loss)
